# Initial kernel scaffold; baseline (speedup 1.0000x reference)
import jax
import jax.numpy as jnp
from jax.experimental import pallas as pl


def kernel(x_0, x_t, t, q_mats, q_one_step_transposed):
    raise NotImplementedError("write your pallas kernel here")


# speedup vs baseline: 1.4901x; 1.4901x over previous
import functools

import jax
import jax.numpy as jnp
from jax import lax
from jax.experimental import pallas as pl
from jax.experimental.pallas import tpu as pltpu
from jax.experimental.pallas import tpu_sc as plsc

K = 512
T_MAX = 100
EPS = 1e-6
B = 16
S = 2048

NW = 32
TOK = B * S
TPW = TOK // NW
C = 64
NCHUNK = TPW // C


def _prep_body(tb1_ref, tb2_ref, t_ref, q1_ref, q2_ref, lt1_ref, lt2_ref):
    b = pl.program_id(0)
    tval = t_ref[b]
    z = 1.0 + K * EPS

    q1 = q1_ref[0]
    lt1 = jnp.log(q1 + EPS)
    lt1 = jnp.where(tval == 1, jnp.zeros_like(lt1), lt1)
    lt1_ref[0] = lt1

    q2 = q2_ref[0]
    colsum = jnp.sum(q2, axis=0, keepdims=True)
    lt2 = jnp.log(q2 + EPS * colsum + EPS * z) - jnp.log(z)
    row = lax.broadcasted_iota(jnp.int32, (K, K), 0)
    col = lax.broadcasted_iota(jnp.int32, (K, K), 1)
    eye_log = jnp.where(row == col, jnp.log(1.0 + EPS), jnp.log(EPS)).astype(jnp.float32)
    lt2 = jnp.where(tval == 1, eye_log, lt2)
    lt2_ref[0] = lt2


def _prep_tables(tb1, tb2, t, q_mats, q_one_step_transposed):
    grid_spec = pltpu.PrefetchScalarGridSpec(
        num_scalar_prefetch=3,
        grid=(B,),
        in_specs=[
            pl.BlockSpec((1, K, K), lambda b, tb1, tb2, t: (tb1[b], 0, 0)),
            pl.BlockSpec((1, K, K), lambda b, tb1, tb2, t: (tb2[b], 0, 0)),
        ],
        out_specs=[
            pl.BlockSpec((1, K, K), lambda b, tb1, tb2, t: (b, 0, 0)),
            pl.BlockSpec((1, K, K), lambda b, tb1, tb2, t: (b, 0, 0)),
        ],
    )
    return pl.pallas_call(
        _prep_body,
        grid_spec=grid_spec,
        out_shape=[
            jax.ShapeDtypeStruct((B, K, K), jnp.float32),
            jax.ShapeDtypeStruct((B, K, K), jnp.float32),
        ],
    )(tb1, tb2, t, q_one_step_transposed, q_mats)


@functools.lru_cache(maxsize=1)
def _make_gather_add():
    mesh = plsc.VectorSubcoreMesh(core_axis_name="c", subcore_axis_name="s")

    @functools.partial(
        pl.kernel,
        mesh=mesh,
        out_type=jax.ShapeDtypeStruct((TOK, K), jnp.float32),
        scratch_types=[
            pltpu.VMEM((C,), jnp.int32),
            pltpu.VMEM((C,), jnp.int32),
            pltpu.VMEM((C, K), jnp.float32),
            pltpu.VMEM((C, K), jnp.float32),
            pltpu.SemaphoreType.DMA,
            pltpu.SemaphoreType.DMA,
        ],
    )
    def _gather_add(lt1_hbm, lt2_hbm, gi1_hbm, gi2_hbm, out_hbm,
                    i1_v, i2_v, r1_v, r2_v, sem1, sem2):
        wid = lax.axis_index("s") * 2 + lax.axis_index("c")
        base = wid * TPW

        def chunk(ci, carry):
            tok = base + ci * C
            pltpu.sync_copy(gi1_hbm.at[pl.ds(tok, C)], i1_v)
            pltpu.sync_copy(gi2_hbm.at[pl.ds(tok, C)], i2_v)
            cp1 = pltpu.async_copy(lt1_hbm.at[i1_v], r1_v, sem1)
            cp2 = pltpu.async_copy(lt2_hbm.at[i2_v], r2_v, sem2)
            cp1.wait()
            cp2.wait()

            def addrow(r, carry2):
                for j in range(K // 16):
                    x = r2_v[r, pl.ds(j * 16, 16)]
                    plsc.addupdate(r1_v.at[r, pl.ds(j * 16, 16)], x)
                return carry2

            lax.fori_loop(0, C, addrow, 0)
            pltpu.sync_copy(r1_v, out_hbm.at[pl.ds(tok, C), :])
            return carry

        lax.fori_loop(0, NCHUNK, chunk, 0)

    return _gather_add


def kernel(x_0, x_t, t, q_mats, q_one_step_transposed):
    t = t.astype(jnp.int32)
    tb1 = (t - 1) % T_MAX
    tb2 = (t - 2) % T_MAX
    lt1, lt2 = _prep_tables(tb1, tb2, t, q_mats, q_one_step_transposed)

    boff = (jnp.arange(B, dtype=jnp.int32) * K)[:, None]
    gi1 = (x_t.astype(jnp.int32) + boff).reshape(-1)
    gi2 = (x_0.astype(jnp.int32) + boff).reshape(-1)

    out = _make_gather_add()(lt1.reshape(B * K, K), lt2.reshape(B * K, K), gi1, gi2)
    return out.reshape(B, S, K)
